# Initial kernel scaffold; baseline (speedup 1.0000x reference)
import jax
import jax.numpy as jnp
from jax.experimental import pallas as pl


def kernel(z_real, z_imag, sym, con):
    raise NotImplementedError("write your pallas kernel here")


# speedup vs baseline: 1.2008x; 1.2008x over previous
import functools

import jax
import jax.numpy as jnp
from jax import lax
from jax.experimental import pallas as pl
from jax.experimental.pallas import tpu as pltpu
from jax.experimental.pallas import tpu_sc as plsc

_B, _T, _DIM = 32, 576, 256
_NSYM, _NCON = 1024, 512
_D = _DIM * 2
_N = _B * _T
_R = 256
_SCALE = 1.25 / (_N * _D)

_DN_T = (((1,), (1,)), ((), ()))


def _stage2_tables_body(sym_ref, con_ref, cn_ref, cit_ref, d2m_ref):
    sym = sym_ref[...]
    sc = lax.dot_general(sym, con_ref[...], _DN_T,
                         preferred_element_type=jnp.float32)
    sn = jnp.sum(sym * sym, axis=1, keepdims=True)
    d2 = (sn + cn_ref[...]) - 2.0 * sc
    m = jnp.min(d2, axis=1, keepdims=True)
    it = lax.broadcasted_iota(jnp.int32, (_NSYM, _NCON), 1)
    cit_ref[...] = jnp.min(jnp.where(d2 == m, it, _NCON), axis=1,
                           keepdims=True)
    d2m_ref[...] = m


def _stage1_body(z_ref, sym_ref, bn_ref, cit_ref, d2m_ref,
                 ohs_ref, ohc_ref, si_ref, ci_ref, sd_ref, cf_ref, dg_ref):
    z = z_ref[...]
    zb = lax.dot_general(z, sym_ref[...], _DN_T,
                         preferred_element_type=jnp.float32)
    rn = jnp.sum(z * z, axis=1, keepdims=True)
    d = (rn + bn_ref[...]) - 2.0 * zb
    mn = jnp.min(d, axis=1, keepdims=True)
    it = lax.broadcasted_iota(jnp.int32, (_R, _NSYM), 1)
    si = jnp.min(jnp.where(d == mn, it, _NSYM), axis=1, keepdims=True)
    oh = it == si
    ohs_ref[...] = oh.astype(jnp.float32)
    ci = jnp.min(jnp.where(oh, cit_ref[...], _NCON), axis=1, keepdims=True)
    dg = jnp.min(jnp.where(oh, d2m_ref[...], jnp.inf), axis=1, keepdims=True)
    it2 = lax.broadcasted_iota(jnp.int32, (_R, _NCON), 1)
    ohc_ref[...] = (it2 == ci).astype(jnp.float32)
    si_ref[...] = si
    ci_ref[...] = ci
    sd_ref[...] = mn
    cf_ref[...] = 1.0 / (1.0 + mn)
    dg_ref[...] = dg


_stage2_tables = pl.pallas_call(
    _stage2_tables_body,
    out_shape=[
        jax.ShapeDtypeStruct((_NSYM, 1), jnp.int32),
        jax.ShapeDtypeStruct((_NSYM, 1), jnp.float32),
    ],
)

_stage1 = pl.pallas_call(
    _stage1_body,
    grid=(_N // _R,),
    in_specs=[
        pl.BlockSpec((_R, _D), lambda i: (i, 0)),
        pl.BlockSpec((_NSYM, _D), lambda i: (0, 0)),
        pl.BlockSpec((1, _NSYM), lambda i: (0, 0)),
        pl.BlockSpec((1, _NSYM), lambda i: (0, 0)),
        pl.BlockSpec((1, _NSYM), lambda i: (0, 0)),
    ],
    out_specs=[
        pl.BlockSpec((_R, _NSYM), lambda i: (i, 0)),
        pl.BlockSpec((_R, _NCON), lambda i: (i, 0)),
        pl.BlockSpec((_R, 1), lambda i: (i, 0)),
        pl.BlockSpec((_R, 1), lambda i: (i, 0)),
        pl.BlockSpec((_R, 1), lambda i: (i, 0)),
        pl.BlockSpec((_R, 1), lambda i: (i, 0)),
        pl.BlockSpec((_R, 1), lambda i: (i, 0)),
    ],
    out_shape=[
        jax.ShapeDtypeStruct((_N, _NSYM), jnp.float32),
        jax.ShapeDtypeStruct((_N, _NCON), jnp.float32),
        jax.ShapeDtypeStruct((_N, 1), jnp.int32),
        jax.ShapeDtypeStruct((_N, 1), jnp.int32),
        jax.ShapeDtypeStruct((_N, 1), jnp.float32),
        jax.ShapeDtypeStruct((_N, 1), jnp.float32),
        jax.ShapeDtypeStruct((_N, 1), jnp.float32),
    ],
    compiler_params=pltpu.CompilerParams(
        dimension_semantics=("parallel",)),
)

_NW = 32
_BPW = _N // _NW
_CH = 96


def _sc_gather_body(table_hbm, idx_hbm, out_hbm, idx_v, rows0, rows1,
                    sem0, sem1):
    wid = lax.axis_index("s") * 2 + lax.axis_index("c")
    base = wid * _BPW
    pltpu.sync_copy(idx_hbm.at[pl.ds(base, _BPW)], idx_v)
    bufs = (rows0, rows1)
    sems = (sem0, sem1)
    nch = _BPW // _CH
    cps = [pltpu.async_copy(
        table_hbm.at[idx_v.at[pl.ds(0, _CH)]], bufs[0], sems[0])]
    for c in range(nch):
        if c + 1 < nch:
            cps.append(pltpu.async_copy(
                table_hbm.at[idx_v.at[pl.ds((c + 1) * _CH, _CH)]],
                bufs[(c + 1) % 2], sems[(c + 1) % 2]))
        cps[c].wait()
        pltpu.sync_copy(bufs[c % 2], out_hbm.at[pl.ds(base + c * _CH, _CH)])


@functools.cache
def _sc_gather():
    return pl.kernel(
        _sc_gather_body,
        out_type=jax.ShapeDtypeStruct((_N, _D), jnp.float32),
        mesh=plsc.VectorSubcoreMesh(core_axis_name="c", subcore_axis_name="s"),
        scratch_types=[
            pltpu.VMEM((_BPW,), jnp.int32),
            pltpu.VMEM((_CH, _D), jnp.float32),
            pltpu.VMEM((_CH, _D), jnp.float32),
            pltpu.SemaphoreType.DMA,
            pltpu.SemaphoreType.DMA,
        ],
    )


def kernel(z_real, z_imag, sym, con):
    zf = jnp.concatenate([z_real, z_imag], axis=-1).reshape(_N, _D)
    bn = jnp.sum(sym**2, axis=-1).reshape(1, _NSYM)
    cn = jnp.sum(con**2, axis=-1).reshape(1, _NCON)
    cit, d2m = _stage2_tables(sym, con, cn)
    ohs, ohc, si2, ci2, sd2, cf2, dg2 = _stage1(
        zf, sym, bn, cit.reshape(1, _NSYM), d2m.reshape(1, _NSYM))
    zs = _sc_gather()(sym, si2.reshape(_N))
    out_c = lax.complex(zs[:, :_DIM], zs[:, _DIM:]).reshape(_B, _T, _DIM)
    ls = _SCALE * jnp.sum(sd2)
    lc = _SCALE * jnp.sum(dg2)
    return (out_c,
            ohs.reshape(_B, _T, _NSYM),
            ohc.reshape(_B, _T, _NCON),
            ls, lc,
            si2.reshape(_B, _T),
            ci2.reshape(_B, _T),
            cf2.reshape(_B, _T))
